# Initial kernel scaffold; baseline (speedup 1.0000x reference)
import jax
import jax.numpy as jnp
from jax.experimental import pallas as pl


def kernel(x, conv_w, gn_w, gn_b, emb):
    raise NotImplementedError("write your pallas kernel here")


# speedup vs baseline: 4.2100x; 4.2100x over previous
import jax
import jax.numpy as jnp
from jax.experimental import pallas as pl

_BS, _L, _DIM = 4, 256, 128
_G = 2
_VAR = _DIM // _G
_C = 512
_EPS = 1e-5


def _vq_kernel(x_ref, w0_ref, w1_ref, e_ref, et_ref, gnw_ref, gnb_ref,
               z_ref, lo_ref, hi_ref):
    xx = x_ref[...]
    e = e_ref[...]
    et = et_ref[...]
    gnw = gnw_ref[...]
    gnb = gnb_ref[...]

    en = jnp.sum(et * et, axis=0, keepdims=True)
    lane_c = jax.lax.broadcasted_iota(jnp.int32, (_L, _C), 1)
    inv_cnt = 1.0 / float(_L * _VAR)

    for g, w_ref in ((0, w0_ref), (1, w1_ref)):
        cols = slice(g * _VAR, (g + 1) * _VAR)
        xg = xx[:, cols]
        wt = w_ref[...]
        y = jnp.dot(xg, wt, preferred_element_type=jnp.float32)

        mean = jnp.sum(y) * inv_cnt
        dcen = y - mean
        var = jnp.sum(dcen * dcen) * inv_cnt
        std = jnp.sqrt(var + _EPS)
        z = dcen / std
        z = z * gnw[:, cols] + gnb[:, cols]

        dot = jnp.dot(z, et, preferred_element_type=jnp.float32)
        zn = jnp.sum(z * z, axis=1, keepdims=True)
        d = jnp.sqrt(jnp.maximum(zn - 2.0 * dot + en, 0.0))

        m1 = jnp.min(d, axis=1, keepdims=True)
        i1 = jnp.min(jnp.where(d == m1, lane_c, _C), axis=1, keepdims=True)
        dmask = jnp.where(lane_c == i1, jnp.inf, d)
        m2 = jnp.min(dmask, axis=1, keepdims=True)
        i2 = jnp.min(jnp.where(dmask == m2, lane_c, _C), axis=1, keepdims=True)

        oh1 = (lane_c == i1).astype(jnp.float32)
        oh2 = (lane_c == i2).astype(jnp.float32)
        e1 = jnp.dot(oh1, e, preferred_element_type=jnp.float32,
                     precision=jax.lax.Precision.HIGHEST)
        e2 = jnp.dot(oh2, e, preferred_element_type=jnp.float32,
                     precision=jax.lax.Precision.HIGHEST)

        swap = i2 < i1
        z_ref[:, cols] = z
        lo_ref[:, cols] = jnp.where(swap, e2, e1)
        hi_ref[:, cols] = jnp.where(swap, e1, e2)


def kernel(x, conv_w, gn_w, gn_b, emb):
    bs, l, d = x.shape
    x2 = x.reshape(bs * l, d)
    w = conv_w[:, :, 0]
    w0t = w[:_VAR, :].T
    w1t = w[_VAR:, :].T
    e = emb[:, 0, :]
    et = e.T
    gnw2 = gn_w.reshape(1, d)
    gnb2 = gn_b.reshape(1, d)

    full = lambda s: pl.BlockSpec(s, lambda b: (0,) * len(s))
    row_block = pl.BlockSpec((l, d), lambda b: (b, 0))
    sh = jax.ShapeDtypeStruct((bs * l, d), jnp.float32)
    z2, lo2, hi2 = pl.pallas_call(
        _vq_kernel,
        grid=(bs,),
        in_specs=[
            row_block,
            full((_VAR, _VAR)),
            full((_VAR, _VAR)),
            full((_C, _VAR)),
            full((_VAR, _C)),
            full((1, d)),
            full((1, d)),
        ],
        out_specs=(row_block, row_block, row_block),
        out_shape=(sh, sh, sh),
    )(x2, w0t, w1t, e, et, gnw2, gnb2)

    ze4 = z2.reshape(bs, l, _G, _VAR)
    cand = jnp.stack([lo2.reshape(bs, l, _G, _VAR),
                      hi2.reshape(bs, l, _G, _VAR)])
    dcand = jnp.linalg.norm(ze4[None] - cand, axis=-1)
    pick_lo = (dcand[0] <= dcand[1])[..., None]
    zq = jnp.where(pick_lo, cand[0], cand[1])
    return zq.reshape(bs, l, d)
